# baseline (speedup 1.0000x reference)
import functools

import jax
import jax.numpy as jnp
from jax import lax
from jax.experimental import pallas as pl
from jax.experimental.pallas import tpu as pltpu
from jax.experimental.pallas import tpu_sc as plsc

_TOPK = 16
_NB = 2048
_W = 2048
_CAP = 256
_LN = 16
_NWORK = 32


def _phase1_body(q_ref, r_ref, q2_ref, r2_ref, dist_ref, t_ref, rmin_ref):
    n = pl.program_id(1)
    nblk = pl.num_programs(1)
    m_rows = q_ref.shape[1]
    nfold = rmin_ref.shape[1]

    q = q_ref[0]
    r = r_ref[0]
    q2 = q2_ref[0, 0]
    r2 = r2_ref[0, 0, 0]

    dot = lax.dot_general(q, r, (((1,), (1,)), ((), ())),
                          preferred_element_type=jnp.float32)
    d2 = (q2[:, None] + r2[None, :]) - 2.0 * dot
    dist_ref[0] = d2

    fold = jnp.minimum(d2[:, :nfold], d2[:, nfold:])

    @pl.when(n == 0)
    def _first():
        rmin_ref[...] = fold

    @pl.when(n > 0)
    def _rest():
        rmin_ref[...] = jnp.minimum(rmin_ref[...], fold)

    @pl.when(n == nblk - 1)
    def _threshold():
        rmin = rmin_ref[...]
        bi = lax.broadcasted_iota(jnp.int32, rmin.shape, 1)

        def tbody(_, carry):
            pm, pc = carry
            after = (rmin > pm) | ((rmin == pm) & (bi > pc))
            masked = jnp.where(after, rmin, jnp.inf)
            m = jnp.min(masked, axis=1, keepdims=True)
            c = jnp.min(jnp.where(masked == m, bi, nfold), axis=1,
                        keepdims=True)
            return m, c

        pm, _ = lax.fori_loop(0, _TOPK, tbody,
                              (jnp.full((m_rows, 1), -jnp.inf, jnp.float32),
                               jnp.full((m_rows, 1), -1, jnp.int32)))
        t_ref[0, 0] = pm[:, 0] + (jnp.abs(pm[:, 0]) + 1.0) * 1e-6


def _make_sc_filter(nrows, npad, w, cap):
    nch = npad // w
    rw = nrows // _NWORK
    ngrp = rw // _LN
    mesh = plsc.VectorSubcoreMesh(core_axis_name="c", subcore_axis_name="s",
                                  num_cores=2, num_subcores=16)

    @functools.partial(
        pl.kernel,
        out_type=[jax.ShapeDtypeStruct((nrows * cap,), jnp.float32),
                  jax.ShapeDtypeStruct((nrows * cap,), jnp.int32)],
        mesh=mesh,
        scratch_types=[
            pltpu.VMEM((_LN,), jnp.float32),
            pltpu.VMEM((2, _LN, w), jnp.float32),
            pltpu.VMEM((_LN * cap,), jnp.float32),
            pltpu.VMEM((_LN * cap,), jnp.int32),
            pltpu.VMEM((_LN,), jnp.int32),
            pltpu.SemaphoreType.DMA,
        ],
        compiler_params=pltpu.CompilerParams(needs_layout_passes=False),
    )
    def sc_filter(dist_hbm, t_hbm, cval_hbm, cidx_hbm, tv, buf, cvb, cib,
                  curs, sem):
        wid = lax.axis_index("s") * 2 + lax.axis_index("c")
        lanes = lax.iota(jnp.int32, _LN)
        inf16 = jnp.full((_LN,), jnp.inf, jnp.float32)
        grp = 1024
        nacc = 4
        cap16 = jnp.full((_LN,), cap, jnp.int32)

        def rowgroup(g, _unused):
            row0 = wid * rw + g * _LN
            pltpu.sync_copy(t_hbm.at[pl.ds(row0, _LN)], tv)

            def initcol(j, carry):
                cvb[pl.ds(j * _LN, _LN)] = inf16
                return carry

            lax.fori_loop(0, cap, initcol, 0)
            curs[...] = jnp.zeros((_LN,), jnp.int32)

            def fire(c, parity):
                pltpu.async_copy(
                    dist_hbm.at[pl.ds(row0, _LN), pl.ds(c * w, w)],
                    buf.at[parity], sem)

            def drain(c, parity):
                pltpu.make_async_copy(
                    dist_hbm.at[pl.ds(row0, _LN), pl.ds(c * w, w)],
                    buf.at[parity], sem).wait()

            fire(0, 0)

            def chunk(c, _c):
                parity = lax.rem(c, 2)

                @pl.when(c + 1 < nch)
                def _prefetch():
                    fire(c + 1, 1 - parity)

                drain(c, parity)

                def row(i, _r):
                    isp = jnp.full((_LN,), i, jnp.int32)
                    tsp = plsc.load_gather(tv, [isp])
                    cur = plsc.load_gather(curs, [isp])
                    rowbase = i * cap

                    def load(j):
                        return buf[parity, i, pl.ds(j, _LN)]

                    def colgrp(gg, cur):
                        j0 = gg * grp
                        accs = [inf16] * nacc
                        for k in range(grp // _LN):
                            accs[k % nacc] = jnp.minimum(
                                accs[k % nacc], load(j0 + k * _LN))
                        acc = accs[0]
                        for k in range(1, nacc):
                            acc = jnp.minimum(acc, accs[k])

                        def hit(cur):
                            for s in range(grp // 128):
                                sacc = load(j0 + s * 128)
                                for k in range(1, 8):
                                    sacc = jnp.minimum(
                                        sacc, load(j0 + s * 128 + k * _LN))

                                def hit2(cur):
                                    for k in range(8):
                                        j = j0 + s * 128 + k * _LN
                                        v = load(j)
                                        mk = v <= tsp
                                        pos = plsc.cumsum(
                                            mk.astype(jnp.int32))
                                        off = cur + pos - 1
                                        mk = mk & (off < cap16)
                                        addr = rowbase + off
                                        colv = (c * w + j) + lanes
                                        plsc.store_scatter(cvb, [addr], v,
                                                           mask=mk)
                                        plsc.store_scatter(cib, [addr], colv,
                                                           mask=mk)
                                        pcnt = (
                                            plsc.all_reduce_population_count(
                                                v <= tsp))
                                        cur = jnp.minimum(cur + pcnt, cap16)
                                    return cur

                                cur = lax.cond(jnp.any(sacc <= tsp), hit2,
                                               lambda cur: cur, cur)
                            return cur

                        return lax.cond(jnp.any(acc <= tsp), hit,
                                        lambda cur: cur, cur)

                    cur = lax.fori_loop(0, w // grp, colgrp, cur)
                    plsc.store_scatter(curs, [isp], cur,
                                       mask=lanes == jnp.int32(0))
                    return _r

                lax.fori_loop(0, _LN, row, 0)
                return _c

            lax.fori_loop(0, nch, chunk, 0)
            pltpu.sync_copy(cvb, cval_hbm.at[pl.ds(row0 * cap, _LN * cap)])
            pltpu.sync_copy(cib, cidx_hbm.at[pl.ds(row0 * cap, _LN * cap)])
            return _unused

        lax.fori_loop(0, ngrp, rowgroup, 0)

    return sc_filter


def _phase3_body(cv_ref, ci_ref, dval_ref, didx_ref):
    cv = jnp.sqrt(jnp.maximum(cv_ref[...], 0.0))
    ci = ci_ref[...]
    nrows = cv.shape[0]
    pm = jnp.full((nrows, 1), -jnp.inf, jnp.float32)
    pc = jnp.full((nrows, 1), -1, jnp.int32)
    big = jnp.int32(2**30)
    ms, cs = [], []
    for _ in range(_TOPK):
        after = (cv > pm) | ((cv == pm) & (ci > pc))
        masked = jnp.where(after, cv, jnp.inf)
        pm = jnp.min(masked, axis=1, keepdims=True)
        pc = jnp.min(jnp.where(masked == pm, ci, big), axis=1, keepdims=True)
        ms.append(pm)
        cs.append(pc)
    dval_ref[...] = jnp.concatenate(ms, axis=1)
    didx_ref[...] = jnp.concatenate(cs, axis=1)


def kernel(ref, query):
    B, N, D = ref.shape
    M = query.shape[1]
    nblk = -(-N // _NB)
    npad = nblk * _NB
    nrows = B * M

    r2 = jnp.sum(ref * ref, axis=2)
    q2 = jnp.sum(query * query, axis=2)
    refp = ref
    r2p = r2
    if npad != N:
        refp = jnp.concatenate(
            [ref, jnp.zeros((B, npad - N, D), ref.dtype)], axis=1)
        r2p = jnp.concatenate(
            [r2, jnp.full((B, npad - N), 1e30, r2.dtype)], axis=1)
    r2p = r2p.reshape(B, nblk, 1, _NB)
    q2r = q2.reshape(B, 1, M)

    phase1 = pl.pallas_call(
        _phase1_body,
        grid=(1, nblk),
        in_specs=[
            pl.BlockSpec((1, M, D), lambda b, n: (b, 0, 0)),
            pl.BlockSpec((1, _NB, D), lambda b, n: (b, n, 0)),
            pl.BlockSpec((1, 1, M), lambda b, n: (b, 0, 0)),
            pl.BlockSpec((1, 1, 1, _NB), lambda b, n: (b, n, 0, 0)),
        ],
        out_specs=[
            pl.BlockSpec((1, M, _NB), lambda b, n: (b, 0, n)),
            pl.BlockSpec((1, 1, M), lambda b, n: (b, 0, 0)),
        ],
        out_shape=[
            jax.ShapeDtypeStruct((1, M, npad), jnp.float32),
            jax.ShapeDtypeStruct((1, 1, M), jnp.float32),
        ],
        scratch_shapes=[pltpu.VMEM((M, _NB // 2), jnp.float32)],
        compiler_params=pltpu.CompilerParams(
            dimension_semantics=("arbitrary", "arbitrary")),
    )
    sc_filter = _make_sc_filter(M, npad, _W, _CAP)
    phase3 = pl.pallas_call(
        _phase3_body,
        out_shape=[
            jax.ShapeDtypeStruct((M, _TOPK), jnp.float32),
            jax.ShapeDtypeStruct((M, _TOPK), jnp.int32),
        ],
    )

    dvs, dis = [], []
    for b in range(B):
        dist, t = phase1(query[b:b + 1], refp[b:b + 1], q2r[b:b + 1],
                         r2p[b:b + 1])
        cval, cidx = sc_filter(dist.reshape(M, npad), t.reshape(M))
        dv, di = phase3(cval.reshape(M, _CAP), cidx.reshape(M, _CAP))
        dvs.append(dv)
        dis.append(di)
    return (jnp.stack(dvs, axis=0),
            jnp.stack(dis, axis=0).astype(jnp.int64))


# speedup vs baseline: 1.2142x; 1.2142x over previous
import functools

import jax
import jax.numpy as jnp
from jax import lax
from jax.experimental import pallas as pl
from jax.experimental.pallas import tpu as pltpu
from jax.experimental.pallas import tpu_sc as plsc

_TOPK = 16
_NB = 2048
_W = 2048
_CAP = 256
_LN = 16
_NWORK = 32


def _phase1_body(q_ref, r_ref, q2_ref, r2_ref, dist_ref, t_ref, rmin_ref):
    n = pl.program_id(1)
    nblk = pl.num_programs(1)
    m_rows = q_ref.shape[1]
    nfold = rmin_ref.shape[1]

    q = q_ref[0]
    r = r_ref[0]
    q2 = q2_ref[0, 0]
    r2 = r2_ref[0, 0, 0]

    dot = lax.dot_general(q, r, (((1,), (1,)), ((), ())),
                          preferred_element_type=jnp.float32)
    d2 = (q2[:, None] + r2[None, :]) - 2.0 * dot
    dist_ref[0] = d2

    fold = jnp.minimum(d2[:, :nfold], d2[:, nfold:])

    @pl.when(n == 0)
    def _first():
        rmin_ref[...] = fold

    @pl.when(n > 0)
    def _rest():
        rmin_ref[...] = jnp.minimum(rmin_ref[...], fold)

    @pl.when(n == nblk - 1)
    def _threshold():
        rmin = rmin_ref[...]
        bi = lax.broadcasted_iota(jnp.int32, rmin.shape, 1)

        def tbody(_, carry):
            pm, pc = carry
            after = (rmin > pm) | ((rmin == pm) & (bi > pc))
            masked = jnp.where(after, rmin, jnp.inf)
            m = jnp.min(masked, axis=1, keepdims=True)
            c = jnp.min(jnp.where(masked == m, bi, nfold), axis=1,
                        keepdims=True)
            return m, c

        pm, _ = lax.fori_loop(0, _TOPK, tbody,
                              (jnp.full((m_rows, 1), -jnp.inf, jnp.float32),
                               jnp.full((m_rows, 1), -1, jnp.int32)))
        t_ref[0, 0] = pm[:, 0] + (jnp.abs(pm[:, 0]) + 1.0) * 1e-6


def _make_sc_filter(nrows, npad, w, cap):
    nch = npad // w
    rw = nrows // _NWORK
    ngrp = rw // _LN
    mesh = plsc.VectorSubcoreMesh(core_axis_name="c", subcore_axis_name="s",
                                  num_cores=2, num_subcores=16)

    @functools.partial(
        pl.kernel,
        out_type=[jax.ShapeDtypeStruct((nrows * cap,), jnp.float32),
                  jax.ShapeDtypeStruct((nrows * cap,), jnp.int32)],
        mesh=mesh,
        scratch_types=[
            pltpu.VMEM((_LN,), jnp.float32),
            pltpu.VMEM((2, _LN, w), jnp.float32),
            pltpu.VMEM((_LN * cap,), jnp.float32),
            pltpu.VMEM((_LN * cap,), jnp.int32),
            pltpu.VMEM((_LN,), jnp.int32),
            pltpu.SemaphoreType.DMA,
        ],
        compiler_params=pltpu.CompilerParams(needs_layout_passes=False),
    )
    def sc_filter(dist_hbm, t_hbm, cval_hbm, cidx_hbm, tv, buf, cvb, cib,
                  curs, sem):
        wid = lax.axis_index("s") * 2 + lax.axis_index("c")
        lanes = lax.iota(jnp.int32, _LN)
        inf16 = jnp.full((_LN,), jnp.inf, jnp.float32)
        grp = 256
        nacc = 4
        cap16 = jnp.full((_LN,), cap, jnp.int32)

        def rowgroup(g, _unused):
            row0 = wid * rw + g * _LN
            pltpu.sync_copy(t_hbm.at[pl.ds(row0, _LN)], tv)

            def initcol(j, carry):
                cvb[pl.ds(j * _LN, _LN)] = inf16
                return carry

            lax.fori_loop(0, cap, initcol, 0)
            curs[...] = jnp.zeros((_LN,), jnp.int32)

            def fire(c, parity):
                pltpu.async_copy(
                    dist_hbm.at[pl.ds(row0, _LN), pl.ds(c * w, w)],
                    buf.at[parity], sem)

            def drain(c, parity):
                pltpu.make_async_copy(
                    dist_hbm.at[pl.ds(row0, _LN), pl.ds(c * w, w)],
                    buf.at[parity], sem).wait()

            fire(0, 0)

            def chunk(c, _c):
                parity = lax.rem(c, 2)

                @pl.when(c + 1 < nch)
                def _prefetch():
                    fire(c + 1, 1 - parity)

                drain(c, parity)

                def row(i, _r):
                    isp = jnp.full((_LN,), i, jnp.int32)
                    tsp = plsc.load_gather(tv, [isp])
                    cur = plsc.load_gather(curs, [isp])
                    rowbase = i * cap

                    def load(j):
                        return buf[parity, i, pl.ds(j, _LN)]

                    def colgrp(gg, cur):
                        j0 = gg * grp
                        accs = [inf16] * nacc
                        for k in range(grp // _LN):
                            accs[k % nacc] = jnp.minimum(
                                accs[k % nacc], load(j0 + k * _LN))
                        acc = accs[0]
                        for k in range(1, nacc):
                            acc = jnp.minimum(acc, accs[k])

                        def hit(cur):
                            for s in range(grp // 128):
                                sacc = load(j0 + s * 128)
                                for k in range(1, 8):
                                    sacc = jnp.minimum(
                                        sacc, load(j0 + s * 128 + k * _LN))

                                def hit2(cur):
                                    for k in range(8):
                                        j = j0 + s * 128 + k * _LN
                                        v = load(j)
                                        mk = v <= tsp
                                        pos = plsc.cumsum(
                                            mk.astype(jnp.int32))
                                        off = cur + pos - 1
                                        mk = mk & (off < cap16)
                                        addr = rowbase + off
                                        colv = (c * w + j) + lanes
                                        plsc.store_scatter(cvb, [addr], v,
                                                           mask=mk)
                                        plsc.store_scatter(cib, [addr], colv,
                                                           mask=mk)
                                        pcnt = (
                                            plsc.all_reduce_population_count(
                                                v <= tsp))
                                        cur = jnp.minimum(cur + pcnt, cap16)
                                    return cur

                                cur = lax.cond(jnp.any(sacc <= tsp), hit2,
                                               lambda cur: cur, cur)
                            return cur

                        return lax.cond(jnp.any(acc <= tsp), hit,
                                        lambda cur: cur, cur)

                    cur = lax.fori_loop(0, w // grp, colgrp, cur)
                    plsc.store_scatter(curs, [isp], cur,
                                       mask=lanes == jnp.int32(0))
                    return _r

                lax.fori_loop(0, _LN, row, 0)
                return _c

            lax.fori_loop(0, nch, chunk, 0)
            pltpu.sync_copy(cvb, cval_hbm.at[pl.ds(row0 * cap, _LN * cap)])
            pltpu.sync_copy(cib, cidx_hbm.at[pl.ds(row0 * cap, _LN * cap)])
            return _unused

        lax.fori_loop(0, ngrp, rowgroup, 0)

    return sc_filter


def _phase3_body(cv_ref, ci_ref, dval_ref, didx_ref):
    cv = jnp.sqrt(jnp.maximum(cv_ref[...], 0.0))
    ci = ci_ref[...]
    nrows = cv.shape[0]
    pm = jnp.full((nrows, 1), -jnp.inf, jnp.float32)
    pc = jnp.full((nrows, 1), -1, jnp.int32)
    big = jnp.int32(2**30)
    ms, cs = [], []
    for _ in range(_TOPK):
        after = (cv > pm) | ((cv == pm) & (ci > pc))
        masked = jnp.where(after, cv, jnp.inf)
        pm = jnp.min(masked, axis=1, keepdims=True)
        pc = jnp.min(jnp.where(masked == pm, ci, big), axis=1, keepdims=True)
        ms.append(pm)
        cs.append(pc)
    dval_ref[...] = jnp.concatenate(ms, axis=1)
    didx_ref[...] = jnp.concatenate(cs, axis=1)


def kernel(ref, query):
    B, N, D = ref.shape
    M = query.shape[1]
    nblk = -(-N // _NB)
    npad = nblk * _NB
    nrows = B * M

    r2 = jnp.sum(ref * ref, axis=2)
    q2 = jnp.sum(query * query, axis=2)
    refp = ref
    r2p = r2
    if npad != N:
        refp = jnp.concatenate(
            [ref, jnp.zeros((B, npad - N, D), ref.dtype)], axis=1)
        r2p = jnp.concatenate(
            [r2, jnp.full((B, npad - N), 1e30, r2.dtype)], axis=1)
    r2p = r2p.reshape(B, nblk, 1, _NB)
    q2r = q2.reshape(B, 1, M)

    phase1 = pl.pallas_call(
        _phase1_body,
        grid=(1, nblk),
        in_specs=[
            pl.BlockSpec((1, M, D), lambda b, n: (b, 0, 0)),
            pl.BlockSpec((1, _NB, D), lambda b, n: (b, n, 0)),
            pl.BlockSpec((1, 1, M), lambda b, n: (b, 0, 0)),
            pl.BlockSpec((1, 1, 1, _NB), lambda b, n: (b, n, 0, 0)),
        ],
        out_specs=[
            pl.BlockSpec((1, M, _NB), lambda b, n: (b, 0, n)),
            pl.BlockSpec((1, 1, M), lambda b, n: (b, 0, 0)),
        ],
        out_shape=[
            jax.ShapeDtypeStruct((1, M, npad), jnp.float32),
            jax.ShapeDtypeStruct((1, 1, M), jnp.float32),
        ],
        scratch_shapes=[pltpu.VMEM((M, _NB // 2), jnp.float32)],
        compiler_params=pltpu.CompilerParams(
            dimension_semantics=("arbitrary", "arbitrary")),
    )
    sc_filter = _make_sc_filter(M, npad, _W, _CAP)
    phase3 = pl.pallas_call(
        _phase3_body,
        out_shape=[
            jax.ShapeDtypeStruct((M, _TOPK), jnp.float32),
            jax.ShapeDtypeStruct((M, _TOPK), jnp.int32),
        ],
    )

    dvs, dis = [], []
    for b in range(B):
        dist, t = phase1(query[b:b + 1], refp[b:b + 1], q2r[b:b + 1],
                         r2p[b:b + 1])
        cval, cidx = sc_filter(dist.reshape(M, npad), t.reshape(M))
        dv, di = phase3(cval.reshape(M, _CAP), cidx.reshape(M, _CAP))
        dvs.append(dv)
        dis.append(di)
    return (jnp.stack(dvs, axis=0),
            jnp.stack(dis, axis=0).astype(jnp.int64))


# speedup vs baseline: 1.5799x; 1.3012x over previous
import functools

import jax
import jax.numpy as jnp
from jax import lax
from jax.experimental import pallas as pl
from jax.experimental.pallas import tpu as pltpu
from jax.experimental.pallas import tpu_sc as plsc

_TOPK = 16
_NB = 2048
_W = 2048
_CAP = 256
_LN = 16
_NWORK = 32


def _phase1_body(q_ref, r_ref, q2_ref, r2_ref, dist_ref, t_ref, rmin_ref):
    n = pl.program_id(1)
    nblk = pl.num_programs(1)
    m_rows = q_ref.shape[1]
    nfold = rmin_ref.shape[1]

    q = q_ref[0]
    r = r_ref[0]
    q2 = q2_ref[0, 0]
    r2 = r2_ref[0, 0, 0]

    dot = lax.dot_general(q, r, (((1,), (1,)), ((), ())),
                          preferred_element_type=jnp.float32)
    d2 = (q2[:, None] + r2[None, :]) - 2.0 * dot
    dist_ref[0] = d2

    fold = jnp.minimum(d2[:, :nfold], d2[:, nfold:])

    @pl.when(n == 0)
    def _first():
        rmin_ref[...] = fold

    @pl.when(n > 0)
    def _rest():
        rmin_ref[...] = jnp.minimum(rmin_ref[...], fold)

    @pl.when(n == nblk - 1)
    def _threshold():
        rmin = rmin_ref[...]
        bi = lax.broadcasted_iota(jnp.int32, rmin.shape, 1)

        def tbody(_, carry):
            pm, pc = carry
            after = (rmin > pm) | ((rmin == pm) & (bi > pc))
            masked = jnp.where(after, rmin, jnp.inf)
            m = jnp.min(masked, axis=1, keepdims=True)
            c = jnp.min(jnp.where(masked == m, bi, nfold), axis=1,
                        keepdims=True)
            return m, c

        pm, _ = lax.fori_loop(0, _TOPK, tbody,
                              (jnp.full((m_rows, 1), -jnp.inf, jnp.float32),
                               jnp.full((m_rows, 1), -1, jnp.int32)))
        t_ref[0, 0] = pm[:, 0] + (jnp.abs(pm[:, 0]) + 1.0) * 1e-6


def _make_sc_filter(nrows, npad, w, cap):
    nch = npad // w
    rw = nrows // _NWORK
    ngrp = rw // _LN
    mesh = plsc.VectorSubcoreMesh(core_axis_name="c", subcore_axis_name="s",
                                  num_cores=2, num_subcores=16)

    @functools.partial(
        pl.kernel,
        out_type=[jax.ShapeDtypeStruct((nrows * cap,), jnp.float32),
                  jax.ShapeDtypeStruct((nrows * cap,), jnp.int32)],
        mesh=mesh,
        scratch_types=[
            pltpu.VMEM((_LN,), jnp.float32),
            pltpu.VMEM((2, _LN, w), jnp.float32),
            pltpu.VMEM((_LN * cap,), jnp.float32),
            pltpu.VMEM((_LN * cap,), jnp.int32),
            pltpu.VMEM((_LN,), jnp.int32),
            pltpu.SemaphoreType.DMA,
        ],
        compiler_params=pltpu.CompilerParams(needs_layout_passes=False),
    )
    def sc_filter(dist_hbm, t_hbm, cval_hbm, cidx_hbm, tv, buf, cvb, cib,
                  curs, sem):
        wid = lax.axis_index("s") * 2 + lax.axis_index("c")
        lanes = lax.iota(jnp.int32, _LN)
        inf16 = jnp.full((_LN,), jnp.inf, jnp.float32)
        grp = 512
        nacc = 4
        cap16 = jnp.full((_LN,), cap, jnp.int32)

        def rowgroup(g, _unused):
            row0 = wid * rw + g * _LN
            pltpu.sync_copy(t_hbm.at[pl.ds(row0, _LN)], tv)

            def initcol(j, carry):
                cvb[pl.ds(j * _LN, _LN)] = inf16
                return carry

            lax.fori_loop(0, cap, initcol, 0)
            curs[...] = jnp.zeros((_LN,), jnp.int32)

            def fire(c, parity):
                pltpu.async_copy(
                    dist_hbm.at[pl.ds(row0, _LN), pl.ds(c * w, w)],
                    buf.at[parity], sem)

            def drain(c, parity):
                pltpu.make_async_copy(
                    dist_hbm.at[pl.ds(row0, _LN), pl.ds(c * w, w)],
                    buf.at[parity], sem).wait()

            fire(0, 0)

            def chunk(c, _c):
                parity = lax.rem(c, 2)

                @pl.when(c + 1 < nch)
                def _prefetch():
                    fire(c + 1, 1 - parity)

                drain(c, parity)

                def row(i, _r):
                    isp = jnp.full((_LN,), i, jnp.int32)
                    tsp = plsc.load_gather(tv, [isp])
                    cur = plsc.load_gather(curs, [isp])
                    rowbase = i * cap

                    def load(j):
                        return buf[parity, i, pl.ds(j, _LN)]

                    def colgrp(gg, cur):
                        j0 = gg * grp
                        accs = [inf16] * nacc
                        for k in range(grp // _LN):
                            accs[k % nacc] = jnp.minimum(
                                accs[k % nacc], load(j0 + k * _LN))
                        acc = accs[0]
                        for k in range(1, nacc):
                            acc = jnp.minimum(acc, accs[k])

                        def hit(cur):
                            for s in range(grp // 128):
                                sacc = load(j0 + s * 128)
                                for k in range(1, 8):
                                    sacc = jnp.minimum(
                                        sacc, load(j0 + s * 128 + k * _LN))

                                def hit2(cur):
                                    for k in range(8):
                                        j = j0 + s * 128 + k * _LN
                                        v = load(j)
                                        mk = v <= tsp
                                        pos = plsc.cumsum(
                                            mk.astype(jnp.int32))
                                        off = cur + pos - 1
                                        mk = mk & (off < cap16)
                                        addr = rowbase + off
                                        colv = (c * w + j) + lanes
                                        plsc.store_scatter(cvb, [addr], v,
                                                           mask=mk)
                                        plsc.store_scatter(cib, [addr], colv,
                                                           mask=mk)
                                        pcnt = (
                                            plsc.all_reduce_population_count(
                                                v <= tsp))
                                        cur = jnp.minimum(cur + pcnt, cap16)
                                    return cur

                                cur = lax.cond(jnp.any(sacc <= tsp), hit2,
                                               lambda cur: cur, cur)
                            return cur

                        return lax.cond(jnp.any(acc <= tsp), hit,
                                        lambda cur: cur, cur)

                    cur = lax.fori_loop(0, w // grp, colgrp, cur)
                    plsc.store_scatter(curs, [isp], cur,
                                       mask=lanes == jnp.int32(0))
                    return _r

                lax.fori_loop(0, _LN, row, 0)
                return _c

            lax.fori_loop(0, nch, chunk, 0)
            pltpu.sync_copy(cvb, cval_hbm.at[pl.ds(row0 * cap, _LN * cap)])
            pltpu.sync_copy(cib, cidx_hbm.at[pl.ds(row0 * cap, _LN * cap)])
            return _unused

        lax.fori_loop(0, ngrp, rowgroup, 0)

    return sc_filter


def _phase3_body(cv_ref, ci_ref, dval_ref, didx_ref):
    cv = jnp.sqrt(jnp.maximum(cv_ref[...], 0.0))
    ci = ci_ref[...]
    nrows = cv.shape[0]
    pm = jnp.full((nrows, 1), -jnp.inf, jnp.float32)
    pc = jnp.full((nrows, 1), -1, jnp.int32)
    big = jnp.int32(2**30)
    ms, cs = [], []
    for _ in range(_TOPK):
        after = (cv > pm) | ((cv == pm) & (ci > pc))
        masked = jnp.where(after, cv, jnp.inf)
        pm = jnp.min(masked, axis=1, keepdims=True)
        pc = jnp.min(jnp.where(masked == pm, ci, big), axis=1, keepdims=True)
        ms.append(pm)
        cs.append(pc)
    dval_ref[...] = jnp.concatenate(ms, axis=1)
    didx_ref[...] = jnp.concatenate(cs, axis=1)


def kernel(ref, query):
    B, N, D = ref.shape
    M = query.shape[1]
    nblk = -(-N // _NB)
    npad = nblk * _NB
    nrows = B * M

    r2 = jnp.sum(ref * ref, axis=2)
    q2 = jnp.sum(query * query, axis=2)
    refp = ref
    r2p = r2
    if npad != N:
        refp = jnp.concatenate(
            [ref, jnp.zeros((B, npad - N, D), ref.dtype)], axis=1)
        r2p = jnp.concatenate(
            [r2, jnp.full((B, npad - N), 1e30, r2.dtype)], axis=1)
    r2p = r2p.reshape(B, nblk, 1, _NB)
    q2r = q2.reshape(B, 1, M)

    mh = M // 2
    phase1 = pl.pallas_call(
        _phase1_body,
        grid=(1, nblk),
        in_specs=[
            pl.BlockSpec((1, mh, D), lambda b, n: (b, 0, 0)),
            pl.BlockSpec((1, _NB, D), lambda b, n: (b, n, 0)),
            pl.BlockSpec((1, 1, mh), lambda b, n: (b, 0, 0)),
            pl.BlockSpec((1, 1, 1, _NB), lambda b, n: (b, n, 0, 0)),
        ],
        out_specs=[
            pl.BlockSpec((1, mh, _NB), lambda b, n: (b, 0, n)),
            pl.BlockSpec((1, 1, mh), lambda b, n: (b, 0, 0)),
        ],
        out_shape=[
            jax.ShapeDtypeStruct((1, mh, npad), jnp.float32),
            jax.ShapeDtypeStruct((1, 1, mh), jnp.float32),
        ],
        scratch_shapes=[pltpu.VMEM((mh, _NB // 2), jnp.float32)],
        compiler_params=pltpu.CompilerParams(
            dimension_semantics=("arbitrary", "arbitrary")),
    )
    sc_filter = _make_sc_filter(mh, npad, _W, _CAP)
    phase3 = pl.pallas_call(
        _phase3_body,
        out_shape=[
            jax.ShapeDtypeStruct((mh, _TOPK), jnp.float32),
            jax.ShapeDtypeStruct((mh, _TOPK), jnp.int32),
        ],
    )

    dvs, dis = [], []
    for b in range(B):
        for h in range(2):
            dist, t = phase1(query[b:b + 1, h * mh:(h + 1) * mh],
                             refp[b:b + 1],
                             q2r[b:b + 1, :, h * mh:(h + 1) * mh],
                             r2p[b:b + 1])
            cval, cidx = sc_filter(dist.reshape(mh, npad), t.reshape(mh))
            dv, di = phase3(cval.reshape(mh, _CAP), cidx.reshape(mh, _CAP))
            dvs.append(dv)
            dis.append(di)
    dval = jnp.concatenate(dvs, axis=0).reshape(B, M, _TOPK)
    didx = jnp.concatenate(dis, axis=0).reshape(B, M, _TOPK)
    return dval, didx.astype(jnp.int64)
